# baseline (speedup 1.0000x reference)
import functools

import jax
import jax.numpy as jnp
from jax import lax
from jax.experimental import pallas as pl
from jax.experimental.pallas import tpu as pltpu
from jax.experimental.pallas import tpu_sc as plsc

DIM = 64
B = 4096 * 50
NC = 2
NS = 16
NW = NC * NS
B_PER_W = B // NW
NBUF = 4
CHUNK = 400
NCHUNK = B_PER_W // CHUNK

_mesh = plsc.VectorSubcoreMesh(core_axis_name="c", subcore_axis_name="s")


@functools.partial(
    pl.kernel,
    mesh=_mesh,
    out_type=jax.ShapeDtypeStruct((B, DIM), jnp.float32),
    compiler_params=pltpu.CompilerParams(use_tc_tiling_on_sc=False),
    scratch_types=[
        pltpu.VMEM((B_PER_W,), jnp.int32),
        [pltpu.VMEM((CHUNK, DIM), jnp.float32) for _ in range(NBUF)],
        [pltpu.SemaphoreType.DMA for _ in range(NBUF)],
        [pltpu.SemaphoreType.DMA for _ in range(NBUF)],
    ],
)
def _embed(idx_hbm, table_hbm, out_hbm, idx_v, bufs, gsems, wsems):
    wid = lax.axis_index("s") * NC + lax.axis_index("c")
    base = wid * B_PER_W
    pltpu.sync_copy(idx_hbm.at[pl.ds(base, B_PER_W)], idx_v)

    gcp = [None] * NBUF
    wcp = [None] * NBUF
    for c in range(NCHUNK):
        b = c % NBUF
        if wcp[b] is not None:
            wcp[b].wait()
        gcp[b] = pltpu.async_copy(
            table_hbm.at[idx_v.at[pl.ds(c * CHUNK, CHUNK)]],
            bufs[b],
            gsems[b],
        )
        if c >= 1:
            bb = (c - 1) % NBUF
            gcp[bb].wait()
            wcp[bb] = pltpu.async_copy(
                bufs[bb],
                out_hbm.at[pl.ds(base + (c - 1) * CHUNK, CHUNK)],
                wsems[bb],
            )
    last = (NCHUNK - 1) % NBUF
    gcp[last].wait()
    wcp[last] = pltpu.async_copy(
        bufs[last],
        out_hbm.at[pl.ds(base + (NCHUNK - 1) * CHUNK, CHUNK)],
        wsems[last],
    )
    for b in range(NBUF):
        if wcp[b] is not None:
            wcp[b].wait()


def kernel(x, table):
    idx = x.reshape(-1).astype(jnp.int32)
    out = _embed(idx, table)
    return out.reshape(x.shape + (DIM,))


# speedup vs baseline: 1.9773x; 1.9773x over previous
import functools

import jax
import jax.numpy as jnp
from jax import lax
from jax.experimental import pallas as pl
from jax.experimental.pallas import tpu as pltpu
from jax.experimental.pallas import tpu_sc as plsc

VOC = 100000
DIM = 64
SEQ = 50
BN = 4096
NC = 2
NS = 16
NW = NC * NS
D_PER_W = DIM // NW

_mesh = plsc.VectorSubcoreMesh(core_axis_name="c", subcore_axis_name="s")


@functools.partial(
    pl.kernel,
    mesh=_mesh,
    out_type=jax.ShapeDtypeStruct((SEQ, DIM, BN), jnp.float32),
    compiler_params=pltpu.CompilerParams(needs_layout_passes=False),
    scratch_types=[
        pltpu.VMEM((VOC,), jnp.float32),
        [pltpu.VMEM((BN,), jnp.int32) for _ in range(2)],
        [pltpu.VMEM((BN,), jnp.float32) for _ in range(2)],
        pltpu.SemaphoreType.DMA,
        [pltpu.SemaphoreType.DMA for _ in range(2)],
        [pltpu.SemaphoreType.DMA for _ in range(2)],
    ],
)
def _embed(xt_hbm, tablet_hbm, out_hbm, trow, xrows, orows, tsem, xsems, wsems):
    wid = lax.axis_index("s") * NC + lax.axis_index("c")
    wcp = [None, None]
    for dd in range(D_PER_W):
        d = wid * D_PER_W + dd
        tcp = pltpu.async_copy(tablet_hbm.at[d], trow, tsem)
        xcp = [None, None]
        xcp[0] = pltpu.async_copy(xt_hbm.at[0], xrows[0], xsems[0])
        tcp.wait()
        for t in range(SEQ):
            b = t % 2
            if t + 1 < SEQ:
                xcp[1 - b] = pltpu.async_copy(
                    xt_hbm.at[t + 1], xrows[1 - b], xsems[1 - b]
                )
            xcp[b].wait()
            if wcp[b] is not None:
                wcp[b].wait()
            xrow = xrows[b]
            orow = orows[b]

            @plsc.parallel_loop(0, BN, 16, unroll=8)
            def _gather(j):
                idx = xrow[pl.ds(j, 16)]
                orow[pl.ds(j, 16)] = plsc.load_gather(trow, [idx])

            wcp[b] = pltpu.async_copy(orow, out_hbm.at[t, d], wsems[b])
    wcp[0].wait()
    wcp[1].wait()


def kernel(x, table):
    out_t = _embed(x.T, table.T)
    return jnp.transpose(out_t, (2, 0, 1))


# speedup vs baseline: 3.0770x; 1.5562x over previous
import functools

import jax
import jax.numpy as jnp
from jax import lax
from jax.experimental import pallas as pl
from jax.experimental.pallas import tpu as pltpu
from jax.experimental.pallas import tpu_sc as plsc

VOC = 100000
DIM = 64
SEQ = 50
BN = 4096
NC = 2
NS = 16
NW = NC * NS
D_PER_W = DIM // NW

_mesh = plsc.VectorSubcoreMesh(core_axis_name="c", subcore_axis_name="s")


@functools.partial(
    pl.kernel,
    mesh=_mesh,
    out_type=jax.ShapeDtypeStruct((SEQ, DIM, BN), jnp.float32),
    compiler_params=pltpu.CompilerParams(needs_layout_passes=False),
    scratch_types=[
        pltpu.VMEM((VOC,), jnp.float32),
        [pltpu.VMEM((BN,), jnp.int32) for _ in range(2)],
        [pltpu.VMEM((BN,), jnp.float32) for _ in range(2)],
        pltpu.VMEM_SHARED((SEQ * BN,), jnp.int32),
        pltpu.SemaphoreType.DMA,
        [pltpu.SemaphoreType.DMA for _ in range(2)],
        [pltpu.SemaphoreType.DMA for _ in range(2)],
    ],
)
def _embed(xt_hbm, tablet_hbm, out_hbm, trow, xrows, orows, x_sp, tsem, xsems, wsems):
    sid = lax.axis_index("s")
    wid = sid * NC + lax.axis_index("c")

    for k in range((SEQ + NS - 1) // NS):
        t_load = k * NS + sid

        @pl.when(t_load < SEQ)
        def _load_x():
            pltpu.sync_copy(xt_hbm.at[t_load], x_sp.at[pl.ds(t_load * BN, BN)])

    plsc.subcore_barrier()

    wcp = [None, None]
    for dd in range(D_PER_W):
        d = wid * D_PER_W + dd
        tcp = pltpu.async_copy(tablet_hbm.at[d], trow, tsem)
        xcp = [None, None]
        xcp[0] = pltpu.async_copy(x_sp.at[pl.ds(0, BN)], xrows[0], xsems[0])
        tcp.wait()
        for t in range(SEQ):
            b = t % 2
            if t + 1 < SEQ:
                xcp[1 - b] = pltpu.async_copy(
                    x_sp.at[pl.ds((t + 1) * BN, BN)], xrows[1 - b], xsems[1 - b]
                )
            xcp[b].wait()
            if wcp[b] is not None:
                wcp[b].wait()
            xrow = xrows[b]
            orow = orows[b]

            @plsc.parallel_loop(0, BN, 16, unroll=8)
            def _gather(j):
                idx = xrow[pl.ds(j, 16)]
                orow[pl.ds(j, 16)] = plsc.load_gather(trow, [idx])

            wcp[b] = pltpu.async_copy(orow, out_hbm.at[t, d], wsems[b])
    wcp[0].wait()
    wcp[1].wait()


def kernel(x, table):
    out_t = _embed(x.T, table.T)
    return jnp.transpose(out_t, (2, 0, 1))
